# Initial kernel scaffold; baseline (speedup 1.0000x reference)
import jax
import jax.numpy as jnp
from jax.experimental import pallas as pl


def kernel(x, attr, mask, W, b):
    raise NotImplementedError("write your pallas kernel here")


# speedup vs baseline: 123.3723x; 123.3723x over previous
import functools
import jax
import jax.numpy as jnp
from jax.experimental import pallas as pl
from jax.experimental.pallas import tpu as pltpu

_PCT = 0.05
_PATCH_VAL = 0.0


def _infid_block(x_ref, attr_ref, W_ref, b_ref, out_ref):
    Bb, n = x_ref.shape
    m = W_ref.shape[1]
    ps = int(n * _PCT)
    P = n // ps
    STEPS = P + 2
    BC = Bb * m

    x = x_ref[...]
    s = jnp.sign(x)
    bias = b_ref[...]

    a_cols = []
    v_cols = []
    for p in range(P):
        sl = slice(p * ps, (p + 1) * ps)
        t = jnp.maximum(attr_ref[:, :, sl] * s[:, None, sl], 0.0)
        a_cols.append(jnp.sum(t, axis=-1, keepdims=True))
        v_cols.append(
            jax.lax.dot_general(
                x[:, sl], W_ref[sl, :],
                (((1,), (0,)), ((), ())),
                preferred_element_type=jnp.float32,
            )[:, None, :]
        )
    a = jnp.concatenate(a_cols, axis=-1)
    v = jnp.concatenate(v_cols, axis=1)
    z0 = jnp.sum(v, axis=1) + bias

    af = a.reshape(BC, P)
    pidx = jax.lax.broadcasted_iota(jnp.int32, (P, P), 0)
    qidx = jax.lax.broadcasted_iota(jnp.int32, (P, P), 1)
    ap = af[:, :, None]
    aq = af[:, None, :]
    beats = (aq > ap) | ((aq == ap) & (qidx < pidx)[None])
    rank = jnp.sum(beats.astype(jnp.int32), axis=-1)

    iidx = jax.lax.broadcasted_iota(jnp.int32, (P, P), 0)
    M = (rank[:, None, :] <= iidx[None, :, :]).astype(jnp.float32)
    v2 = jnp.broadcast_to(v[:, None, :, :], (Bb, m, P, m)).reshape(BC, P, m)
    S = jax.lax.dot_general(
        M, v2, (((2,), (1,)), ((0,), (0,))),
        preferred_element_type=jnp.float32,
    )

    z0b = jnp.broadcast_to(z0[:, None, :], (Bb, m, m)).reshape(BC, m)
    zmid = z0b[:, None, :] - S
    z_first = z0b[:, None, :]
    z_last = jnp.broadcast_to(bias[:, None, :], (BC, 1, m))
    L = jnp.concatenate([z_first, zmid, z_last], axis=1)

    L = L - jnp.max(L, axis=-1, keepdims=True)
    e = jnp.exp(L)
    pr = e / jnp.sum(e, axis=-1, keepdims=True)

    ci = jax.lax.broadcasted_iota(jnp.int32, (m, m), 0)
    ki = jax.lax.broadcasted_iota(jnp.int32, (m, m), 1)
    eye = (ci == ki).astype(jnp.float32)
    eyeb = jnp.broadcast_to(eye[None, :, :], (Bb, m, m)).reshape(BC, m)
    Pc = jnp.sum(pr * eyeb[:, None, :], axis=-1)
    r = Pc / Pc[:, 0:1]
    out = (jnp.sum(r, axis=-1, keepdims=True)
           - 0.5 * (r[:, 0:1] + r[:, STEPS - 1:STEPS]))
    out_ref[...] = out * (1.0 / STEPS)


@jax.jit
def kernel(x, attr, mask, W, b):
    B, m, n = attr.shape
    del mask
    BB = 8
    grid = (B // BB,)
    out = pl.pallas_call(
        _infid_block,
        grid=grid,
        in_specs=[
            pl.BlockSpec((BB, n), lambda i: (i, 0)),
            pl.BlockSpec((BB, m, n), lambda i: (i, 0, 0)),
            pl.BlockSpec((n, m), lambda i: (0, 0)),
            pl.BlockSpec((1, m), lambda i: (0, 0)),
        ],
        out_specs=pl.BlockSpec((BB * m, 1), lambda i: (i, 0)),
        out_shape=jax.ShapeDtypeStruct((B * m, 1), jnp.float32),
    )(x, attr, W, b.reshape(1, m))
    return out.reshape(B, m)


# speedup vs baseline: 151.0910x; 1.2247x over previous
import functools
import jax
import jax.numpy as jnp
from jax import lax
from jax.experimental import pallas as pl
from jax.experimental.pallas import tpu as pltpu
from jax.experimental.pallas import tpu_sc as plsc

_PCT = 0.05
_PATCH_VAL = 0.0
_NEG = -1e30


def _prep_block(x_ref, attr_ref, W_ref, b_ref, a_ref, v_ref, z0_ref):
    Bb, n = x_ref.shape
    m = W_ref.shape[1]
    ps = int(n * _PCT)
    P = n // ps

    x = x_ref[...]
    s = jnp.sign(x)
    bias = b_ref[...]

    a_cols = []
    v_cols = []
    for p in range(P):
        sl = slice(p * ps, (p + 1) * ps)
        t = jnp.maximum(attr_ref[:, :, sl] * s[:, None, sl], 0.0)
        a_cols.append(jnp.sum(t, axis=-1, keepdims=True))
        v_cols.append(
            jax.lax.dot_general(
                x[:, sl], W_ref[sl, :],
                (((1,), (0,)), ((), ())),
                preferred_element_type=jnp.float32,
            )[:, None, :]
        )
    a = jnp.concatenate(a_cols, axis=-1)
    v = jnp.concatenate(v_cols, axis=1)
    z0 = jnp.sum(v, axis=1) + bias

    a_ref[...] = jnp.concatenate(
        [a, jnp.full((Bb, m, 32 - P), _NEG, jnp.float32)], axis=-1)
    v_ref[...] = jnp.concatenate(
        [v, jnp.zeros((Bb, P, 16 - m), jnp.float32)], axis=-1)
    z0_ref[:, 0, :] = jnp.concatenate(
        [z0, jnp.full((Bb, 16 - m), _NEG, jnp.float32)], axis=-1)


def _take(vec, idx):
    dnums = lax.GatherDimensionNumbers(
        offset_dims=(), collapsed_slice_dims=(0,), start_index_map=(0,))
    return lax.gather(vec, idx[:, None], dnums, (1,),
                      mode=lax.GatherScatterMode.PROMISE_IN_BOUNDS)


def _make_sc_stage(B, m, P):
    info = plsc.get_sparse_core_info()
    NW = info.num_cores * info.num_subcores
    bpw = B // NW
    STEPS = P + 2
    mesh = plsc.VectorSubcoreMesh(core_axis_name="c", subcore_axis_name="s")

    @functools.partial(
        pl.kernel, mesh=mesh,
        out_type=jax.ShapeDtypeStruct((B, m, 16), jnp.float32),
        scratch_types=[
            pltpu.VMEM((bpw, m, 32), jnp.float32),
            pltpu.VMEM((bpw, P, 16), jnp.float32),
            pltpu.VMEM((bpw, 1, 16), jnp.float32),
            pltpu.VMEM((16,), jnp.float32),
            pltpu.VMEM((bpw, m, 16), jnp.float32),
        ],
    )
    def sc_fn(a_hbm, v_hbm, z0_hbm, bias_hbm, out_hbm,
              a_v, v_v, z0_v, b_v, o_v):
        wid = lax.axis_index("s") * info.num_cores + lax.axis_index("c")
        base = wid * bpw
        pltpu.sync_copy(a_hbm.at[pl.ds(base, bpw)], a_v)
        pltpu.sync_copy(v_hbm.at[pl.ds(base, bpw)], v_v)
        pltpu.sync_copy(z0_hbm.at[pl.ds(base, bpw)], z0_v)
        pltpu.sync_copy(bias_hbm, b_v)

        iota = lax.iota(jnp.int32, 16)
        zeros_i = jnp.zeros((16,), jnp.int32)

        def splat_sum(e):
            for sh in (8, 4, 2, 1):
                e = e + _take(e, iota ^ sh)
            return e

        bias_vec = b_v[...]
        eb = jnp.exp(bias_vec)
        p_last = eb / splat_sum(eb)

        for b_local in range(bpw):
            z0_vec = z0_v[b_local, 0, :]
            e0 = jnp.exp(z0_vec)
            p0 = e0 / splat_sum(e0)

            def task(cc, carry):
                a_lo = a_v[b_local, cc, pl.ds(0, 16)]
                a_hi = a_v[b_local, cc, pl.ds(16, 16)]
                rank_lo = zeros_i
                rank_hi = zeros_i
                for q in range(P):
                    if q < 16:
                        aq = _take(a_lo, jnp.full((16,), q, jnp.int32))
                    else:
                        aq = _take(a_hi, jnp.full((16,), q - 16, jnp.int32))
                    blo = (aq > a_lo) | ((aq == a_lo) & (q < iota))
                    bhi = (aq > a_hi) | ((aq == a_hi) & ((q - 16) < iota))
                    rank_lo = rank_lo + jnp.where(blo, 1, 0)
                    rank_hi = rank_hi + jnp.where(bhi, 1, 0)
                acc = z0_vec
                psum = jnp.zeros((16,), jnp.float32)
                for i in range(P):
                    mv = (jnp.where(rank_lo == i, iota, 0)
                          + jnp.where(rank_hi == i, iota + 16, 0))
                    for sh in (8, 4, 2, 1):
                        mv = mv + _take(mv, iota ^ sh)
                    pi = lax.squeeze(lax.slice_in_dim(mv, 0, 1), (0,))
                    row = v_v[b_local, pi, :]
                    acc = acc - row
                    e = jnp.exp(acc)
                    psum = psum + e / splat_sum(e)
                psum = psum + 0.5 * p_last
                result = (0.5 + psum / p0) * (1.0 / STEPS)
                o_v[b_local, cc, :] = result
                return carry

            lax.fori_loop(0, m, task, 0)

        pltpu.sync_copy(o_v, out_hbm.at[pl.ds(base, bpw)])

    return sc_fn


@jax.jit
def kernel(x, attr, mask, W, b):
    B, m, n = attr.shape
    del mask
    ps = int(n * _PCT)
    P = n // ps
    BB = 8
    grid = (B // BB,)
    a_pad, v16, z016 = pl.pallas_call(
        _prep_block,
        grid=grid,
        in_specs=[
            pl.BlockSpec((BB, n), lambda i: (i, 0)),
            pl.BlockSpec((BB, m, n), lambda i: (i, 0, 0)),
            pl.BlockSpec((n, m), lambda i: (0, 0)),
            pl.BlockSpec((1, m), lambda i: (0, 0)),
        ],
        out_specs=[
            pl.BlockSpec((BB, m, 32), lambda i: (i, 0, 0)),
            pl.BlockSpec((BB, P, 16), lambda i: (i, 0, 0)),
            pl.BlockSpec((BB, 1, 16), lambda i: (i, 0, 0)),
        ],
        out_shape=[
            jax.ShapeDtypeStruct((B, m, 32), jnp.float32),
            jax.ShapeDtypeStruct((B, P, 16), jnp.float32),
            jax.ShapeDtypeStruct((B, 1, 16), jnp.float32),
        ],
    )(x, attr, W, b.reshape(1, m))

    bias16 = jnp.concatenate([b, jnp.full((16 - m,), _NEG, jnp.float32)])
    out3 = _make_sc_stage(B, m, P)(a_pad, v16, z016, bias16)
    idx = jnp.arange(m)
    return out3[:, idx, idx]


# speedup vs baseline: 172.4548x; 1.1414x over previous
import functools
import jax
import jax.numpy as jnp
import numpy as np
from jax import lax
from jax.experimental import pallas as pl
from jax.experimental.pallas import tpu as pltpu
from jax.experimental.pallas import tpu_sc as plsc

_PCT = 0.05
_PATCH_VAL = 0.0
_NEG = -1e30


def _prep_block(x_ref, attr_ref, W_ref, b_ref, E_ref,
                a_ref, v_ref, z0_ref, pl_ref):
    Bb, n = x_ref.shape
    m = W_ref.shape[1]
    ps = int(n * _PCT)
    P = n // ps

    x = x_ref[...]
    s = jnp.sign(x)
    bias = b_ref[...]

    t = jnp.maximum(attr_ref[...] * s[:, None, :], 0.0)
    af = jax.lax.dot_general(
        t.reshape(Bb * m, n), E_ref[...],
        (((1,), (0,)), ((), ())),
        preferred_element_type=jnp.float32,
    )
    a = af.reshape(Bb, m, P)
    a_ref[...] = jnp.concatenate(
        [a, jnp.full((Bb, m, 32 - P), _NEG, jnp.float32)], axis=-1)

    v_cols = []
    for p in range(P):
        sl = slice(p * ps, (p + 1) * ps)
        v_cols.append(
            jax.lax.dot_general(
                x[:, sl], W_ref[sl, :],
                (((1,), (0,)), ((), ())),
                preferred_element_type=jnp.float32,
            )[:, None, :]
        )
    v = jnp.concatenate(v_cols, axis=1)
    z0 = jnp.sum(v, axis=1) + bias

    v_ref[...] = jnp.concatenate(
        [v, jnp.zeros((Bb, P, 16 - m), jnp.float32)], axis=-1)
    z0_ref[:, 0, :] = jnp.concatenate(
        [z0, jnp.full((Bb, 16 - m), _NEG, jnp.float32)], axis=-1)

    eb = jnp.exp(bias)
    plast = eb / jnp.sum(eb, axis=-1, keepdims=True)
    pl_ref[...] = jnp.concatenate(
        [plast, jnp.zeros((1, 16 - m), jnp.float32)], axis=-1)


def _take(vec, idx):
    dnums = lax.GatherDimensionNumbers(
        offset_dims=(), collapsed_slice_dims=(0,), start_index_map=(0,))
    return lax.gather(vec, idx[:, None], dnums, (1,),
                      mode=lax.GatherScatterMode.PROMISE_IN_BOUNDS)


def _make_sc_stage(B, m, P):
    info = plsc.get_sparse_core_info()
    NW = info.num_cores * info.num_subcores
    bpw = B // NW
    STEPS = P + 2
    mesh = plsc.VectorSubcoreMesh(core_axis_name="c", subcore_axis_name="s")

    @functools.partial(
        pl.kernel, mesh=mesh,
        out_type=jax.ShapeDtypeStruct((B, 1, 16), jnp.float32),
        scratch_types=[
            pltpu.VMEM((bpw, m, 32), jnp.float32),
            pltpu.VMEM((bpw, P, 16), jnp.float32),
            pltpu.VMEM((bpw, 1, 16), jnp.float32),
            pltpu.VMEM((1, 16), jnp.float32),
            pltpu.VMEM((bpw, 1, 16), jnp.float32),
        ],
    )
    def sc_fn(a_hbm, v_hbm, z0_hbm, plast_hbm, out_hbm,
              a_v, v_v, z0_v, p_v, o_v):
        wid = lax.axis_index("s") * info.num_cores + lax.axis_index("c")
        base = wid * bpw
        pltpu.sync_copy(a_hbm.at[pl.ds(base, bpw)], a_v)
        pltpu.sync_copy(v_hbm.at[pl.ds(base, bpw)], v_v)
        pltpu.sync_copy(z0_hbm.at[pl.ds(base, bpw)], z0_v)
        pltpu.sync_copy(plast_hbm, p_v)

        iota = lax.iota(jnp.int32, 16)
        zeros_i = jnp.zeros((16,), jnp.int32)

        def splat_sum(e):
            for sh in (8, 4, 2, 1):
                e = e + _take(e, iota ^ sh)
            return e

        p_last = p_v[0, :]

        for b_local in range(bpw):
            z0_vec = z0_v[b_local, 0, :]
            e0 = jnp.exp(z0_vec)
            p0 = e0 / splat_sum(e0)

            def task(cc, out_acc):
                a_lo = a_v[b_local, cc, pl.ds(0, 16)]
                a_hi = a_v[b_local, cc, pl.ds(16, 16)]
                rank_lo = zeros_i
                rank_hi = zeros_i
                for q in range(P):
                    if q < 16:
                        aq = _take(a_lo, jnp.full((16,), q, jnp.int32))
                    else:
                        aq = _take(a_hi, jnp.full((16,), q - 16, jnp.int32))
                    blo = (aq > a_lo) | ((aq == a_lo) & (q < iota))
                    bhi = (aq > a_hi) | ((aq == a_hi) & ((q - 16) < iota))
                    rank_lo = rank_lo + jnp.where(blo, 1, 0)
                    rank_hi = rank_hi + jnp.where(bhi, 1, 0)
                acc = z0_vec
                psum = jnp.zeros((16,), jnp.float32)
                for i in range(P):
                    mv = (jnp.where(rank_lo == i, iota, 0)
                          + jnp.where(rank_hi == i, iota + 16, 0))
                    for sh in (8, 4, 2, 1):
                        mv = mv + _take(mv, iota ^ sh)
                    pi = lax.squeeze(lax.slice_in_dim(mv, 0, 1), (0,))
                    row = v_v[b_local, pi, :]
                    acc = acc - row
                    e = jnp.exp(acc)
                    psum = psum + e / splat_sum(e)
                psum = psum + 0.5 * p_last
                result = (0.5 + psum / p0) * (1.0 / STEPS)
                return out_acc + jnp.where(iota == cc, result, 0.0)

            out_vec = lax.fori_loop(0, m, task, jnp.zeros((16,), jnp.float32))
            o_v[b_local, 0, :] = out_vec

        pltpu.sync_copy(o_v, out_hbm.at[pl.ds(base, bpw)])

    return sc_fn


@jax.jit
def kernel(x, attr, mask, W, b):
    B, m, n = attr.shape
    del mask
    ps = int(n * _PCT)
    P = n // ps
    BB = 8
    grid = (B // BB,)
    E = jnp.asarray(np.repeat(np.eye(P, dtype=np.float32), ps, axis=0))
    a_pad, v16, z016, plast16 = pl.pallas_call(
        _prep_block,
        grid=grid,
        in_specs=[
            pl.BlockSpec((BB, n), lambda i: (i, 0)),
            pl.BlockSpec((BB, m, n), lambda i: (i, 0, 0)),
            pl.BlockSpec((n, m), lambda i: (0, 0)),
            pl.BlockSpec((1, m), lambda i: (0, 0)),
            pl.BlockSpec((n, P), lambda i: (0, 0)),
        ],
        out_specs=[
            pl.BlockSpec((BB, m, 32), lambda i: (i, 0, 0)),
            pl.BlockSpec((BB, P, 16), lambda i: (i, 0, 0)),
            pl.BlockSpec((BB, 1, 16), lambda i: (i, 0, 0)),
            pl.BlockSpec((1, 16), lambda i: (0, 0)),
        ],
        out_shape=[
            jax.ShapeDtypeStruct((B, m, 32), jnp.float32),
            jax.ShapeDtypeStruct((B, P, 16), jnp.float32),
            jax.ShapeDtypeStruct((B, 1, 16), jnp.float32),
            jax.ShapeDtypeStruct((1, 16), jnp.float32),
        ],
    )(x, attr, W, b.reshape(1, m), E)

    out3 = _make_sc_stage(B, m, P)(a_pad, v16, z016, plast16)
    return out3[:, 0, :m]
